# baseline (speedup 1.0000x reference)
import functools

import jax
import jax.numpy as jnp
from jax import lax
from jax.experimental import pallas as pl
from jax.experimental.pallas import tpu as pltpu
from jax.experimental.pallas import tpu_sc as plsc

_L = 16
_NW = 32
_CHUNK = 64
_GCHUNK = 8
_ROWS_SC = 2048
_PAD_VAL = 1 << 30


@functools.partial(jax.jit, static_argnames=("n", "d", "idxp"))
def _sc_unpool(h, g, idx_pad, n, d, idxp):
    m = g.shape[1]
    t_all = n // 8
    tpw = t_all // _NW
    extra = t_all % _NW
    maskp = -(-(8 * (tpw + 1)) // _L) * _L
    nchunk = -(-(8 * (tpw + 1)) // _CHUNK)
    g_lo = n - _ROWS_SC
    g_per_w = _ROWS_SC // _NW

    mesh = plsc.VectorSubcoreMesh(core_axis_name="c", subcore_axis_name="s")

    @functools.partial(
        pl.kernel,
        out_type=(
            jax.ShapeDtypeStruct((n, d), jnp.float32),
            jax.ShapeDtypeStruct((_ROWS_SC, m), jnp.float32),
        ),
        mesh=mesh,
        scratch_types=[
            pltpu.VMEM((idxp,), jnp.int32),
            pltpu.VMEM((maskp,), jnp.float32),
            pltpu.VMEM((_CHUNK, d), jnp.float32),
            pltpu.VMEM((_GCHUNK, m), jnp.float32),
        ],
        compiler_params=pltpu.CompilerParams(
            needs_layout_passes=False, use_tc_tiling_on_sc=True
        ),
    )
    def k(h_hbm, g_hbm, idx_hbm, out_hbm, gtail_hbm, idx_v, mask_v, buf_v, gbuf_v):
        wid = lax.axis_index("s") * 2 + lax.axis_index("c")
        base = 8 * (tpw * wid + jnp.minimum(wid, extra))
        rows_w = 8 * (tpw + jnp.where(wid < extra, 1, 0))

        zeros16 = jnp.zeros((_L,), jnp.float32)
        ones16 = jnp.ones((_L,), jnp.float32)
        for i in range(maskp // _L):
            mask_v[pl.ds(i * _L, _L)] = zeros16

        pltpu.sync_copy(idx_hbm, idx_v)

        def scat(t, carry):
            v = idx_v[pl.ds(t * _L, _L)]
            local = v - base
            mm = (local >= 0) & (local < maskp)
            plsc.store_scatter(mask_v, [local], ones16, mask=mm)
            return carry

        lax.fori_loop(0, idxp // _L, scat, 0)

        for kk in range(nchunk):
            s = jnp.maximum(jnp.minimum(kk * _CHUNK, rows_w - _CHUNK), 0)
            row0 = pl.multiple_of(base + s, 8)
            pltpu.sync_copy(h_hbm.at[pl.ds(row0, _CHUNK), :], buf_v)

            def grp_body(gi, carry, s=s):
                mvec = mask_v[pl.ds(s + gi * _L, _L)]
                for i in range(_L):
                    mval = mvec[i]
                    r = gi * _L + i
                    for j in range(d // _L):
                        sl = pl.ds(j * _L, _L)
                        buf_v[r, sl] = buf_v[r, sl] * mval
                return carry

            lax.fori_loop(0, _CHUNK // _L, grp_body, 0)
            pltpu.sync_copy(buf_v, out_hbm.at[pl.ds(row0, _CHUNK), :])

        for q in range(g_per_w // _GCHUNK):
            off = pl.multiple_of(wid * g_per_w + q * _GCHUNK, 8)
            src_row = pl.multiple_of(g_lo + wid * g_per_w + q * _GCHUNK, 8)
            pltpu.sync_copy(g_hbm.at[pl.ds(src_row, _GCHUNK), :], gbuf_v)
            pltpu.sync_copy(gbuf_v, gtail_hbm.at[pl.ds(off, _GCHUNK), :])

    return k(h, g, idx_pad)


def _g_top(g, pre_h, rows):
    s = pre_h[0, 0] * jnp.float32(0.0)
    s = jnp.where(jnp.isfinite(s), s, jnp.float32(0.0))
    return g[:rows] + s


def kernel(g, h, pre_h, idx):
    n, d = h.shape
    nidx = idx.shape[0]
    idxp = -(-nidx // _L) * _L
    idx_pad = jnp.full((idxp,), _PAD_VAL, jnp.int32)
    idx_pad = idx_pad.at[:nidx].set(idx.astype(jnp.int32))
    new_h, g_tail = _sc_unpool(h, g, idx_pad, n, d, idxp)
    g_out = jnp.concatenate([_g_top(g, pre_h, n - _ROWS_SC), g_tail], axis=0)
    return (g_out, new_h)


# speedup vs baseline: 1.8594x; 1.8594x over previous
import functools

import jax
import jax.numpy as jnp
from jax import lax
from jax.experimental import pallas as pl
from jax.experimental.pallas import tpu as pltpu
from jax.experimental.pallas import tpu_sc as plsc

_L = 16
_NW = 32
_CHUNK = 128
_PAD_VAL = 1 << 30


@functools.partial(jax.jit, static_argnames=("n", "d", "idxp"))
def _sc_unpool(h, idx_pad, n, d, idxp):
    t_all = n // 8
    tpw = t_all // _NW
    extra = t_all % _NW
    maskp = -(-(8 * (tpw + 1)) // _L) * _L
    nchunk = -(-(8 * (tpw + 1)) // _CHUNK)

    mesh = plsc.VectorSubcoreMesh(core_axis_name="c", subcore_axis_name="s")

    @functools.partial(
        pl.kernel,
        out_type=jax.ShapeDtypeStruct((n, d), jnp.float32),
        mesh=mesh,
        scratch_types=[
            pltpu.VMEM((idxp,), jnp.int32),
            pltpu.VMEM((maskp,), jnp.float32),
            pltpu.VMEM((_CHUNK, d), jnp.float32),
        ],
        compiler_params=pltpu.CompilerParams(
            needs_layout_passes=False, use_tc_tiling_on_sc=True
        ),
    )
    def k(h_hbm, idx_hbm, out_hbm, idx_v, mask_v, buf_v):
        wid = lax.axis_index("s") * 2 + lax.axis_index("c")
        base = 8 * (tpw * wid + jnp.minimum(wid, extra))
        rows_w = 8 * (tpw + jnp.where(wid < extra, 1, 0))

        zeros16 = jnp.zeros((_L,), jnp.float32)
        ones16 = jnp.ones((_L,), jnp.float32)
        for i in range(maskp // _L):
            mask_v[pl.ds(i * _L, _L)] = zeros16

        pltpu.sync_copy(idx_hbm, idx_v)

        def scat(t, carry):
            v = idx_v[pl.ds(t * _L, _L)]
            local = v - base
            m = (local >= 0) & (local < maskp)
            plsc.store_scatter(mask_v, [local], ones16, mask=m)
            return carry

        lax.fori_loop(0, idxp // _L, scat, 0)

        for kk in range(nchunk):
            s = jnp.maximum(jnp.minimum(kk * _CHUNK, rows_w - _CHUNK), 0)
            row0 = pl.multiple_of(base + s, 8)
            pltpu.sync_copy(h_hbm.at[pl.ds(row0, _CHUNK), :], buf_v)

            def grp_body(gi, carry, s=s):
                mvec = mask_v[pl.ds(s + gi * _L, _L)]
                for i in range(_L):
                    m = mvec[i]
                    r = gi * _L + i
                    for j in range(d // _L):
                        sl = pl.ds(j * _L, _L)
                        buf_v[r, sl] = buf_v[r, sl] * m
                return carry

            lax.fori_loop(0, _CHUNK // _L, grp_body, 0)
            pltpu.sync_copy(buf_v, out_hbm.at[pl.ds(row0, _CHUNK), :])

    return k(h, idx_pad)


_CPBLK = 384


_CPNB = 8


def _g_copy(g, pre_h):
    del pre_h
    n, m = g.shape
    blk = 128
    nblk = -(-n // blk)
    starts = [min(i * blk, n - blk) for i in range(nblk)]

    def body(src, dst, *scr):
        bufs = scr[:_CPNB]
        insems = scr[_CPNB:2 * _CPNB]
        outsems = scr[2 * _CPNB:]

        def in_dma(i):
            return pltpu.make_async_copy(
                src.at[pl.ds(starts[i], blk), :], bufs[i % _CPNB],
                insems[i % _CPNB])

        def out_dma(i):
            return pltpu.make_async_copy(
                bufs[i % _CPNB], dst.at[pl.ds(starts[i], blk), :],
                outsems[i % _CPNB])

        for i in range(min(_CPNB, nblk)):
            in_dma(i).start()
        for g0 in range(0, nblk, _CPNB):
            cnt = min(_CPNB, nblk - g0)
            for b in range(cnt):
                in_dma(g0 + b).wait()
                out_dma(g0 + b).start()
            for b in range(cnt):
                out_dma(g0 + b).wait()
                j = g0 + b + _CPNB
                if j < nblk:
                    in_dma(j).start()

    return pl.pallas_call(
        body,
        out_shape=jax.ShapeDtypeStruct(g.shape, g.dtype),
        in_specs=[pl.BlockSpec(memory_space=pl.ANY)],
        out_specs=pl.BlockSpec(memory_space=pl.ANY),
        scratch_shapes=(
            [pltpu.VMEM((blk, m), jnp.float32)] * _CPNB
            + [pltpu.SemaphoreType.DMA] * (2 * _CPNB)
        ),
    )(g)


def kernel(g, h, pre_h, idx):
    n, d = h.shape
    nidx = idx.shape[0]
    idxp = -(-nidx // _L) * _L
    idx_pad = jnp.full((idxp,), _PAD_VAL, jnp.int32)
    idx_pad = idx_pad.at[:nidx].set(idx.astype(jnp.int32))
    new_h = _sc_unpool(h, idx_pad, n, d, idxp)
    return (_g_copy(g, pre_h), new_h)


# speedup vs baseline: 1.8978x; 1.0206x over previous
import functools

import jax
import jax.numpy as jnp
from jax import lax
from jax.experimental import pallas as pl
from jax.experimental.pallas import tpu as pltpu
from jax.experimental.pallas import tpu_sc as plsc

_L = 16
_NW = 32
_CHUNK = 128
_PAD_VAL = 1 << 30


@functools.partial(jax.jit, static_argnames=("n", "d", "nidx"))
def _sc_unpool(h, idx, n, d, nidx):
    idxp = -(-nidx // _L) * _L
    t_all = n // 8
    tpw = t_all // _NW
    extra = t_all % _NW
    maskp = -(-(8 * (tpw + 1)) // _L) * _L
    nchunk = -(-(8 * (tpw + 1)) // _CHUNK)

    mesh = plsc.VectorSubcoreMesh(core_axis_name="c", subcore_axis_name="s")

    @functools.partial(
        pl.kernel,
        out_type=jax.ShapeDtypeStruct((n, d), jnp.float32),
        mesh=mesh,
        scratch_types=[
            pltpu.VMEM((idxp,), jnp.int32),
            pltpu.VMEM((maskp,), jnp.float32),
            pltpu.VMEM((_CHUNK, d), jnp.float32),
        ],
        compiler_params=pltpu.CompilerParams(
            needs_layout_passes=False, use_tc_tiling_on_sc=True
        ),
    )
    def k(h_hbm, idx_hbm, out_hbm, idx_v, mask_v, buf_v):
        wid = lax.axis_index("s") * 2 + lax.axis_index("c")
        base = 8 * (tpw * wid + jnp.minimum(wid, extra))
        rows_w = 8 * (tpw + jnp.where(wid < extra, 1, 0))

        zeros16 = jnp.zeros((_L,), jnp.float32)
        ones16 = jnp.ones((_L,), jnp.float32)
        for i in range(maskp // _L):
            mask_v[pl.ds(i * _L, _L)] = zeros16

        pltpu.sync_copy(idx_hbm, idx_v.at[pl.ds(0, nidx)])
        lane = lax.iota(jnp.int32, _L)

        def scat(t, carry):
            v = idx_v[pl.ds(t * _L, _L)]
            local = v - base
            m = (local >= 0) & (local < maskp) & (t * _L + lane < nidx)
            plsc.store_scatter(mask_v, [local], ones16, mask=m)
            return carry

        lax.fori_loop(0, idxp // _L, scat, 0)

        for kk in range(nchunk):
            s = jnp.maximum(jnp.minimum(kk * _CHUNK, rows_w - _CHUNK), 0)
            row0 = pl.multiple_of(base + s, 8)
            pltpu.sync_copy(h_hbm.at[pl.ds(row0, _CHUNK), :], buf_v)

            def grp_body(gi, carry, s=s):
                mvec = mask_v[pl.ds(s + gi * _L, _L)]
                for i in range(_L):
                    m = mvec[i]
                    r = gi * _L + i
                    for j in range(d // _L):
                        sl = pl.ds(j * _L, _L)
                        buf_v[r, sl] = buf_v[r, sl] * m
                return carry

            lax.fori_loop(0, _CHUNK // _L, grp_body, 0)
            pltpu.sync_copy(buf_v, out_hbm.at[pl.ds(row0, _CHUNK), :])

    return k(h, idx)


_CPBLK = 384


def _g_copy(g, pre_h):
    del pre_h
    n, m = g.shape

    def body(src, dst):
        dst[...] = src[...]

    return pl.pallas_call(
        body,
        grid=(-(-n // _CPBLK),),
        in_specs=[pl.BlockSpec((_CPBLK, m), lambda i: (i, 0))],
        out_specs=pl.BlockSpec((_CPBLK, m), lambda i: (i, 0)),
        out_shape=jax.ShapeDtypeStruct(g.shape, g.dtype),
        compiler_params=pltpu.CompilerParams(
            vmem_limit_bytes=63 * 1024 * 1024
        ),
    )(g)


def kernel(g, h, pre_h, idx):
    n, d = h.shape
    new_h = _sc_unpool(h, idx.astype(jnp.int32), n, d, idx.shape[0])
    return (_g_copy(g, pre_h), new_h)


# speedup vs baseline: 1.8983x; 1.0003x over previous
import functools

import jax
import jax.numpy as jnp
from jax import lax
from jax.experimental import pallas as pl
from jax.experimental.pallas import tpu as pltpu
from jax.experimental.pallas import tpu_sc as plsc

_L = 16
_NW = 32
_CHUNK = 128


@functools.partial(jax.jit, static_argnames=("n", "d", "nidx"))
def _sc_unpool(h, idx, n, d, nidx):
    idxp = -(-nidx // _L) * _L
    t_all = n // 8
    tpw = t_all // _NW
    extra = t_all % _NW
    maskp = -(-(8 * (tpw + 1)) // _L) * _L
    nchunk = -(-(8 * (tpw + 1)) // _CHUNK)

    mesh = plsc.VectorSubcoreMesh(core_axis_name="c", subcore_axis_name="s")

    @functools.partial(
        pl.kernel,
        out_type=jax.ShapeDtypeStruct((n, d), jnp.float32),
        mesh=mesh,
        scratch_types=[
            pltpu.VMEM((idxp,), jnp.int32),
            pltpu.VMEM((maskp,), jnp.float32),
            pltpu.VMEM((_CHUNK, d), jnp.float32),
        ],
        compiler_params=pltpu.CompilerParams(
            needs_layout_passes=False, use_tc_tiling_on_sc=True
        ),
    )
    def k(h_hbm, idx_hbm, out_hbm, idx_v, mask_v, buf_v):
        wid = lax.axis_index("s") * 2 + lax.axis_index("c")
        base = 8 * (tpw * wid + jnp.minimum(wid, extra))
        rows_w = 8 * (tpw + jnp.where(wid < extra, 1, 0))

        zeros16 = jnp.zeros((_L,), jnp.float32)
        ones16 = jnp.ones((_L,), jnp.float32)
        for i in range(maskp // _L):
            mask_v[pl.ds(i * _L, _L)] = zeros16

        pltpu.sync_copy(idx_hbm, idx_v.at[pl.ds(0, nidx)])
        lane = lax.iota(jnp.int32, _L)

        def scat(t, carry):
            v = idx_v[pl.ds(t * _L, _L)]
            local = v - base
            m = (local >= 0) & (local < maskp) & (t * _L + lane < nidx)
            plsc.store_scatter(mask_v, [local], ones16, mask=m)
            return carry

        lax.fori_loop(0, idxp // _L, scat, 0)

        for kk in range(nchunk):
            s = jnp.maximum(jnp.minimum(kk * _CHUNK, rows_w - _CHUNK), 0)
            row0 = pl.multiple_of(base + s, 8)
            pltpu.sync_copy(h_hbm.at[pl.ds(row0, _CHUNK), :], buf_v)

            def grp_body(gi, carry, s=s):
                mvec = mask_v[pl.ds(s + gi * _L, _L)]
                for i in range(_L):
                    m = mvec[i]
                    r = gi * _L + i
                    for j in range(d // _L):
                        sl = pl.ds(j * _L, _L)
                        buf_v[r, sl] = buf_v[r, sl] * m
                return carry

            lax.fori_loop(0, _CHUNK // _L, grp_body, 0)
            pltpu.sync_copy(buf_v, out_hbm.at[pl.ds(row0, _CHUNK), :])

    return k(h, idx)


_CPBLK = 384


def _g_copy(g, pre_h):
    del pre_h
    n, m = g.shape

    def body(src, dst):
        dst[...] = src[...]

    return pl.pallas_call(
        body,
        grid=(-(-n // _CPBLK),),
        in_specs=[pl.BlockSpec((_CPBLK, m), lambda i: (i, 0))],
        out_specs=pl.BlockSpec((_CPBLK, m), lambda i: (i, 0)),
        out_shape=jax.ShapeDtypeStruct(g.shape, g.dtype),
        compiler_params=pltpu.CompilerParams(
            vmem_limit_bytes=63 * 1024 * 1024
        ),
    )(g)


def kernel(g, h, pre_h, idx):
    n, d = h.shape
    new_h = _sc_unpool(h, idx.astype(jnp.int32), n, d, idx.shape[0])
    return (_g_copy(g, pre_h), new_h)


# speedup vs baseline: 1.8991x; 1.0004x over previous
import functools

import jax
import jax.numpy as jnp
from jax import lax
from jax.experimental import pallas as pl
from jax.experimental.pallas import tpu as pltpu
from jax.experimental.pallas import tpu_sc as plsc

_L = 16
_NW = 32
_CHUNK = 128


@functools.partial(jax.jit, static_argnames=("n", "d", "nidx"))
def _sc_unpool(h, idx, n, d, nidx):
    idxp = -(-nidx // _L) * _L
    t_all = n // 8
    tpw = t_all // _NW
    extra = t_all % _NW
    maskp = -(-(8 * (tpw + 1)) // _L) * _L
    nchunk = -(-(8 * (tpw + 1)) // _CHUNK)

    mesh = plsc.VectorSubcoreMesh(core_axis_name="c", subcore_axis_name="s")

    @functools.partial(
        pl.kernel,
        out_type=jax.ShapeDtypeStruct((n, d), jnp.float32),
        mesh=mesh,
        scratch_types=[
            pltpu.VMEM((idxp,), jnp.int32),
            pltpu.VMEM((maskp,), jnp.float32),
            pltpu.VMEM((_CHUNK, d), jnp.float32),
        ],
        compiler_params=pltpu.CompilerParams(
            needs_layout_passes=False, use_tc_tiling_on_sc=True
        ),
    )
    def k(h_hbm, idx_hbm, out_hbm, idx_v, mask_v, buf_v):
        wid = lax.axis_index("s") * 2 + lax.axis_index("c")
        base = 8 * (tpw * wid + jnp.minimum(wid, extra))
        rows_w = 8 * (tpw + jnp.where(wid < extra, 1, 0))

        zeros16 = jnp.zeros((_L,), jnp.float32)
        ones16 = jnp.ones((_L,), jnp.float32)
        for i in range(maskp // _L):
            mask_v[pl.ds(i * _L, _L)] = zeros16

        pltpu.sync_copy(idx_hbm, idx_v.at[pl.ds(0, nidx)])
        lane = lax.iota(jnp.int32, _L)

        def scat(t, carry):
            v = idx_v[pl.ds(t * _L, _L)]
            local = v - base
            m = (local >= 0) & (local < maskp) & (t * _L + lane < nidx)
            plsc.store_scatter(mask_v, [local], ones16, mask=m)
            return carry

        lax.fori_loop(0, idxp // _L, scat, 0)

        for kk in range(nchunk):
            s = jnp.maximum(jnp.minimum(kk * _CHUNK, rows_w - _CHUNK), 0)
            row0 = pl.multiple_of(base + s, 8)
            pltpu.sync_copy(h_hbm.at[pl.ds(row0, _CHUNK), :], buf_v)

            def grp_body(gi, carry, s=s):
                mvec = mask_v[pl.ds(s + gi * _L, _L)]
                for i in range(_L):
                    m = mvec[i]
                    r = gi * _L + i
                    for j in range(d // _L):
                        sl = pl.ds(j * _L, _L)
                        buf_v[r, sl] = buf_v[r, sl] * m
                return carry

            lax.fori_loop(0, _CHUNK // _L, grp_body, 0)
            pltpu.sync_copy(buf_v, out_hbm.at[pl.ds(row0, _CHUNK), :])

    return k(h, idx)


_CPBLK = 320


def _g_copy(g, pre_h):
    del pre_h
    n, m = g.shape

    def body(src, dst):
        dst[...] = src[...]

    return pl.pallas_call(
        body,
        grid=(-(-n // _CPBLK),),
        in_specs=[pl.BlockSpec((_CPBLK, m), lambda i: (i, 0))],
        out_specs=pl.BlockSpec((_CPBLK, m), lambda i: (i, 0)),
        out_shape=jax.ShapeDtypeStruct(g.shape, g.dtype),
        compiler_params=pltpu.CompilerParams(
            vmem_limit_bytes=63 * 1024 * 1024
        ),
    )(g)


def kernel(g, h, pre_h, idx):
    n, d = h.shape
    new_h = _sc_unpool(h, idx.astype(jnp.int32), n, d, idx.shape[0])
    return (_g_copy(g, pre_h), new_h)


# speedup vs baseline: 1.8993x; 1.0001x over previous
import functools

import jax
import jax.numpy as jnp
from jax import lax
from jax.experimental import pallas as pl
from jax.experimental.pallas import tpu as pltpu
from jax.experimental.pallas import tpu_sc as plsc

_L = 16
_NW = 32
_CHUNK = 128


@functools.partial(jax.jit, static_argnames=("n", "d", "nidx"))
def _sc_unpool(h, idx, n, d, nidx):
    idxp = -(-nidx // _L) * _L
    t_all = n // 8
    tpw = t_all // _NW
    extra = t_all % _NW
    maskp = -(-(8 * (tpw + 1)) // _L) * _L
    nchunk = -(-(8 * (tpw + 1)) // _CHUNK)

    mesh = plsc.VectorSubcoreMesh(core_axis_name="c", subcore_axis_name="s")

    @functools.partial(
        pl.kernel,
        out_type=jax.ShapeDtypeStruct((n, d), jnp.float32),
        mesh=mesh,
        scratch_types=[
            pltpu.VMEM((idxp,), jnp.int32),
            pltpu.VMEM((maskp,), jnp.float32),
            pltpu.VMEM((_CHUNK, d), jnp.float32),
        ],
        compiler_params=pltpu.CompilerParams(
            needs_layout_passes=False, use_tc_tiling_on_sc=True
        ),
    )
    def k(h_hbm, idx_hbm, out_hbm, idx_v, mask_v, buf_v):
        wid = lax.axis_index("s") * 2 + lax.axis_index("c")
        base = 8 * (tpw * wid + jnp.minimum(wid, extra))
        rows_w = 8 * (tpw + jnp.where(wid < extra, 1, 0))

        zeros16 = jnp.zeros((_L,), jnp.float32)
        ones16 = jnp.ones((_L,), jnp.float32)
        for i in range(maskp // _L):
            mask_v[pl.ds(i * _L, _L)] = zeros16

        pltpu.sync_copy(idx_hbm, idx_v.at[pl.ds(0, nidx)])
        lane = lax.iota(jnp.int32, _L)

        def scat(t, carry):
            v = idx_v[pl.ds(t * _L, _L)]
            local = v - base
            m = (local >= 0) & (local < maskp) & (t * _L + lane < nidx)
            plsc.store_scatter(mask_v, [local], ones16, mask=m)
            return carry

        lax.fori_loop(0, idxp // _L, scat, 0)

        for kk in range(nchunk):
            s = jnp.maximum(jnp.minimum(kk * _CHUNK, rows_w - _CHUNK), 0)
            row0 = pl.multiple_of(base + s, 8)
            pltpu.sync_copy(h_hbm.at[pl.ds(row0, _CHUNK), :], buf_v)

            def grp_body(gi, carry, s=s):
                mvec = mask_v[pl.ds(s + gi * _L, _L)]
                for i in range(_L):
                    m = mvec[i]
                    r = gi * _L + i
                    for j in range(d // _L):
                        sl = pl.ds(j * _L, _L)
                        buf_v[r, sl] = buf_v[r, sl] * m
                return carry

            lax.fori_loop(0, _CHUNK // _L, grp_body, 0)
            pltpu.sync_copy(buf_v, out_hbm.at[pl.ds(row0, _CHUNK), :])

    return k(h, idx)


_CPBLK = 320


def _g_copy(g):
    n, m = g.shape

    def body(src, dst):
        dst[...] = src[...]

    return pl.pallas_call(
        body,
        grid=(-(-n // _CPBLK),),
        in_specs=[pl.BlockSpec((_CPBLK, m), lambda i: (i, 0))],
        out_specs=pl.BlockSpec((_CPBLK, m), lambda i: (i, 0)),
        out_shape=jax.ShapeDtypeStruct(g.shape, g.dtype),
        compiler_params=pltpu.CompilerParams(
            vmem_limit_bytes=63 * 1024 * 1024
        ),
    )(g)


def kernel(g, h, pre_h, idx):
    n, d = h.shape
    new_h = _sc_unpool(h, idx.astype(jnp.int32), n, d, idx.shape[0])
    return (_g_copy(g), new_h)
